# Initial kernel scaffold; baseline (speedup 1.0000x reference)
import jax
import jax.numpy as jnp
from jax import lax
from jax.experimental import pallas as pl
from jax.experimental.pallas import tpu as pltpu
from jax.experimental.pallas import tpu_sc as plsc

N_NODES = 10000
N_EDGES = 320000
D = 128
DH = 64
NC = 2
NS = 16
L = 16
CHUNK = 80
EPT = N_EDGES // NS
NCH = EPT // CHUNK
ROWS_PT = N_NODES // NS


def _zero_gbuf(gbuf):
    zeros = jnp.zeros((L,), jnp.float32)

    def zb(e, c):
        for d in range(DH // L):
            gbuf[e, pl.ds(d * L, L)] = zeros
        return c

    lax.fori_loop(0, CHUNK, zb, 0)


def _zero_acc(acc, gbuf, sid):
    r0 = sid * ROWS_PT
    full = ROWS_PT // CHUNK
    rem = ROWS_PT - full * CHUNK
    for j in range(full):
        pltpu.sync_copy(gbuf, acc.at[pl.ds(r0 + j * CHUNK, CHUNK)])
    if rem:
        pltpu.sync_copy(gbuf.at[pl.ds(0, rem)],
                        acc.at[pl.ds(r0 + full * CHUNK, rem)])


def _hop(src, dst, colv, rowv, wv, gbuf, sem):
    def chunk_body(i, c):
        pltpu.async_copy(src.at[colv.at[i]], gbuf, sem).wait()

        def scale(e, c2):
            wb = plsc.load_gather(wv, [jnp.full((L,), i, jnp.int32),
                                       jnp.full((L,), e, jnp.int32)])
            for d in range(DH // L):
                sl = pl.ds(d * L, L)
                gbuf[e, sl] = gbuf[e, sl] * wb
            return c2

        lax.fori_loop(0, CHUNK, scale, 0)
        pltpu.sync_copy(gbuf, dst.at[rowv.at[i]], add=True)
        return c

    lax.fori_loop(0, NCH, chunk_body, 0)


def _body(x0, x1, col_r, row_r, w_r, o0, o1,
          acc1, acc2, colv, rowv, wv, gbuf, sem):
    cid = lax.axis_index("c")
    sid = lax.axis_index("s")

    base = sid * NCH
    pltpu.sync_copy(col_r.at[pl.ds(base, NCH)], colv)
    pltpu.sync_copy(row_r.at[pl.ds(base, NCH)], rowv)
    pltpu.sync_copy(w_r.at[pl.ds(base, NCH)], wv)

    _zero_gbuf(gbuf)
    _zero_acc(acc1, gbuf, sid)
    _zero_acc(acc2, gbuf, sid)
    plsc.subcore_barrier()

    @pl.when(cid == 0)
    def _():
        _hop(x0, acc1, colv, rowv, wv, gbuf, sem)

    @pl.when(cid == 1)
    def _():
        _hop(x1, acc1, colv, rowv, wv, gbuf, sem)

    plsc.subcore_barrier()
    _hop(acc1, acc2, colv, rowv, wv, gbuf, sem)
    plsc.subcore_barrier()

    r0 = sid * ROWS_PT

    @pl.when(cid == 0)
    def _():
        pltpu.sync_copy(acc2.at[pl.ds(r0, ROWS_PT)], o0.at[pl.ds(r0, ROWS_PT)])

    @pl.when(cid == 1)
    def _():
        pltpu.sync_copy(acc2.at[pl.ds(r0, ROWS_PT)], o1.at[pl.ds(r0, ROWS_PT)])


def kernel(x, edge_index, edge_values):
    x0 = x[:, :DH]
    x1 = x[:, DH:]
    row_r = edge_index[0].reshape(N_EDGES // CHUNK, CHUNK)
    col_r = edge_index[1].reshape(N_EDGES // CHUNK, CHUNK)
    w_r = edge_values.reshape(N_EDGES // CHUNK, CHUNK)

    f = pl.kernel(
        _body,
        out_type=(jax.ShapeDtypeStruct((N_NODES, DH), jnp.float32),
                  jax.ShapeDtypeStruct((N_NODES, DH), jnp.float32)),
        mesh=plsc.VectorSubcoreMesh(core_axis_name="c", subcore_axis_name="s",
                                    num_cores=NC, num_subcores=NS),
        scratch_types=[
            pltpu.VMEM_SHARED((N_NODES, DH), jnp.float32),
            pltpu.VMEM_SHARED((N_NODES, DH), jnp.float32),
            pltpu.VMEM((NCH, CHUNK), jnp.int32),
            pltpu.VMEM((NCH, CHUNK), jnp.int32),
            pltpu.VMEM((NCH, CHUNK), jnp.float32),
            pltpu.VMEM((CHUNK, DH), jnp.float32),
            pltpu.SemaphoreType.DMA,
        ],
    )
    o0, o1 = f(x0, x1, col_r, row_r, w_r)
    return jnp.concatenate([o0, o1], axis=1)


# speedup vs baseline: 4.3998x; 4.3998x over previous
import jax
import jax.numpy as jnp
from jax import lax
from jax.experimental import pallas as pl
from jax.experimental.pallas import tpu as pltpu
from jax.experimental.pallas import tpu_sc as plsc

N_NODES = 10000
N_EDGES = 320000
D = 128
DH = 64
NC = 2
NS = 16
L = 16
CHUNK = 80
NB = 25
EPT = N_EDGES // NS
NCH = EPT // CHUNK
NSUP = NCH // NB
ROWS_PT = 624
REM_ROWS = N_NODES - NS * ROWS_PT


def _zero_gbuf(gbuf):
    zeros = jnp.zeros((L,), jnp.float32)

    def zb(e, c):
        for d in range(DH // L):
            gbuf[e, pl.ds(d * L, L)] = zeros
        return c

    lax.fori_loop(0, CHUNK, zb, 0)


def _zero_acc(acc, gbuf, sid):
    r0 = pl.multiple_of(sid * ROWS_PT, 8)
    full = ROWS_PT // CHUNK
    rem = ROWS_PT - full * CHUNK
    for j in range(full):
        pltpu.sync_copy(gbuf, acc.at[pl.ds(r0 + j * CHUNK, CHUNK)])
    if rem:
        pltpu.sync_copy(gbuf.at[pl.ds(0, rem)],
                        acc.at[pl.ds(r0 + full * CHUNK, rem)])

    @pl.when(sid == NS - 1)
    def _():
        pltpu.sync_copy(gbuf.at[pl.ds(0, REM_ROWS)],
                        acc.at[pl.ds(NS * ROWS_PT, REM_ROWS)])


def _hop(src, dst, sid, col_r, row_r, w_hbm, colv, rowv, wv, gbuf, sem):
    def super_body(j, c0):
        pltpu.sync_copy(col_r.at[sid, j], colv)
        pltpu.sync_copy(row_r.at[sid, j], rowv)
        wbase = pl.multiple_of(sid * EPT + j * (NB * CHUNK), 8)
        pltpu.sync_copy(w_hbm.at[pl.ds(wbase, NB * CHUNK)], wv)

        def chunk_body(i, c):
            pltpu.async_copy(src.at[colv.at[i]], gbuf, sem).wait()

            def scale(e, c2):
                wb = plsc.load_gather(wv, [jnp.full((L,), i * CHUNK + e,
                                                    jnp.int32)])
                for d in range(DH // L):
                    sl = pl.ds(d * L, L)
                    gbuf[e, sl] = gbuf[e, sl] * wb
                return c2

            lax.fori_loop(0, CHUNK, scale, 0)
            pltpu.sync_copy(gbuf, dst.at[rowv.at[i]], add=True)
            return c

        lax.fori_loop(0, NB, chunk_body, 0)
        return c0

    lax.fori_loop(0, NSUP, super_body, 0)


def _body(x0, x1, col_r, row_r, w_hbm, o0, o1,
          acc1, acc2, colv, rowv, wv, gbuf, sem):
    cid = lax.axis_index("c")
    sid = lax.axis_index("s")

    _zero_gbuf(gbuf)
    _zero_acc(acc1, gbuf, sid)
    _zero_acc(acc2, gbuf, sid)
    plsc.subcore_barrier()

    @pl.when(cid == 0)
    def _():
        _hop(x0, acc1, sid, col_r, row_r, w_hbm, colv, rowv, wv, gbuf, sem)

    @pl.when(cid == 1)
    def _():
        _hop(x1, acc1, sid, col_r, row_r, w_hbm, colv, rowv, wv, gbuf, sem)

    plsc.subcore_barrier()
    _hop(acc1, acc2, sid, col_r, row_r, w_hbm, colv, rowv, wv, gbuf, sem)
    plsc.subcore_barrier()

    r0 = pl.multiple_of(sid * ROWS_PT, 8)
    tail = NS * ROWS_PT

    def _writeback(o):
        pltpu.sync_copy(acc2.at[pl.ds(r0, ROWS_PT)], o.at[pl.ds(r0, ROWS_PT)])

        @pl.when(sid == NS - 1)
        def _():
            pltpu.sync_copy(acc2.at[pl.ds(tail, REM_ROWS)],
                            o.at[pl.ds(tail, REM_ROWS)])

    @pl.when(cid == 0)
    def _():
        _writeback(o0)

    @pl.when(cid == 1)
    def _():
        _writeback(o1)


def kernel(x, edge_index, edge_values):
    x0 = x[:, :DH]
    x1 = x[:, DH:]
    row_r = edge_index[0].reshape(NS, NSUP, NB, CHUNK)
    col_r = edge_index[1].reshape(NS, NSUP, NB, CHUNK)
    w_r = edge_values

    f = pl.kernel(
        _body,
        out_type=(jax.ShapeDtypeStruct((N_NODES, DH), jnp.float32),
                  jax.ShapeDtypeStruct((N_NODES, DH), jnp.float32)),
        mesh=plsc.VectorSubcoreMesh(core_axis_name="c", subcore_axis_name="s",
                                    num_cores=NC, num_subcores=NS),
        compiler_params=pltpu.CompilerParams(needs_layout_passes=False,
                                             use_tc_tiling_on_sc=False),
        scratch_types=[
            pltpu.VMEM_SHARED((N_NODES, DH), jnp.float32),
            pltpu.VMEM_SHARED((N_NODES, DH), jnp.float32),
            pltpu.VMEM((NB, CHUNK), jnp.int32),
            pltpu.VMEM((NB, CHUNK), jnp.int32),
            pltpu.VMEM((NB * CHUNK,), jnp.float32),
            pltpu.VMEM((CHUNK, DH), jnp.float32),
            pltpu.SemaphoreType.DMA,
        ],
    )
    o0, o1 = f(x0, x1, col_r, row_r, w_r)
    return jnp.concatenate([o0, o1], axis=1)


# speedup vs baseline: 8.5990x; 1.9544x over previous
import jax
import jax.numpy as jnp
from jax import lax
from jax.experimental import pallas as pl
from jax.experimental.pallas import tpu as pltpu
from jax.experimental.pallas import tpu_sc as plsc

N_NODES = 10000
N_EDGES = 320000
D = 128
DH = 64
NC = 2
NS = 16
L = 16
CHUNK = 80
NB = 50
EPT = N_EDGES // NS
NCH = EPT // CHUNK
NSUP = NCH // NB
ROWS_PT = 624
REM_ROWS = N_NODES - NS * ROWS_PT


def _zero_buf(buf):
    zeros = jnp.zeros((L,), jnp.float32)

    def zb(e, c):
        for d in range(DH // L):
            buf[e, pl.ds(d * L, L)] = zeros
        return c

    lax.fori_loop(0, CHUNK, zb, 0)


def _zero_acc(acc, zbuf, sid):
    r0 = pl.multiple_of(sid * ROWS_PT, 8)
    full = ROWS_PT // CHUNK
    rem = ROWS_PT - full * CHUNK
    for j in range(full):
        pltpu.sync_copy(zbuf, acc.at[pl.ds(r0 + j * CHUNK, CHUNK)])
    if rem:
        pltpu.sync_copy(zbuf.at[pl.ds(0, rem)],
                        acc.at[pl.ds(r0 + full * CHUNK, rem)])

    @pl.when(sid == NS - 1)
    def _():
        pltpu.sync_copy(zbuf.at[pl.ds(0, REM_ROWS)],
                        acc.at[pl.ds(NS * ROWS_PT, REM_ROWS)])


def _hop(src, dst, sid, col_r, row_r, w_hbm,
         colv, rowv, wv, gbufs, sbufs, gsems, ssems):
    def super_body(j, c0):
        pltpu.sync_copy(col_r.at[sid, j], colv)
        pltpu.sync_copy(row_r.at[sid, j], rowv)
        wbase = pl.multiple_of(sid * EPT + j * (NB * CHUNK), 8)
        pltpu.sync_copy(w_hbm.at[pl.ds(wbase, NB * CHUNK)], wv)

        for b in range(2):
            pltpu.async_copy(src.at[colv.at[b]], gbufs[b], gsems[b])

        def pair_body(k, c):
            for b in range(2):
                i = k * 2 + b
                pltpu.make_async_copy(src.at[colv.at[i]],
                                      gbufs[b], gsems[b]).wait()

                @pl.when(k >= 1)
                def _():
                    pltpu.make_async_copy(sbufs[b], dst.at[rowv.at[i - 2]],
                                          ssems[b]).wait()

                def scale(e, c2):
                    wb = plsc.load_gather(
                        wv, [jnp.full((L,), i * CHUNK + e, jnp.int32)])
                    for d in range(DH // L):
                        sl = pl.ds(d * L, L)
                        sbufs[b][e, sl] = gbufs[b][e, sl] * wb
                    return c2

                lax.fori_loop(0, CHUNK, scale, 0)

                pltpu.async_copy(sbufs[b], dst.at[rowv.at[i]], ssems[b],
                                 add=True)

                @pl.when(k < NB // 2 - 1)
                def _():
                    pltpu.async_copy(src.at[colv.at[i + 2]],
                                     gbufs[b], gsems[b])

            return c

        lax.fori_loop(0, NB // 2, pair_body, 0)

        for b in range(2):
            pltpu.make_async_copy(sbufs[b], dst.at[rowv.at[NB - 2 + b]],
                                  ssems[b]).wait()
        return c0

    lax.fori_loop(0, NSUP, super_body, 0)


def _body(x0, x1, col_r, row_r, w_hbm, o0, o1,
          acc1, acc2, colv, rowv, wv, gbuf0, gbuf1, sbuf0, sbuf1,
          gsem0, gsem1, ssem0, ssem1):
    cid = lax.axis_index("c")
    sid = lax.axis_index("s")
    gbufs = (gbuf0, gbuf1)
    sbufs = (sbuf0, sbuf1)
    gsems = (gsem0, gsem1)
    ssems = (ssem0, ssem1)

    _zero_buf(sbuf0)
    _zero_acc(acc1, sbuf0, sid)
    _zero_acc(acc2, sbuf0, sid)
    plsc.subcore_barrier()

    @pl.when(cid == 0)
    def _():
        _hop(x0, acc1, sid, col_r, row_r, w_hbm,
             colv, rowv, wv, gbufs, sbufs, gsems, ssems)

    @pl.when(cid == 1)
    def _():
        _hop(x1, acc1, sid, col_r, row_r, w_hbm,
             colv, rowv, wv, gbufs, sbufs, gsems, ssems)

    plsc.subcore_barrier()
    _hop(acc1, acc2, sid, col_r, row_r, w_hbm,
         colv, rowv, wv, gbufs, sbufs, gsems, ssems)
    plsc.subcore_barrier()

    r0 = pl.multiple_of(sid * ROWS_PT, 8)
    tail = NS * ROWS_PT

    def _writeback(o):
        pltpu.sync_copy(acc2.at[pl.ds(r0, ROWS_PT)], o.at[pl.ds(r0, ROWS_PT)])

        @pl.when(sid == NS - 1)
        def _():
            pltpu.sync_copy(acc2.at[pl.ds(tail, REM_ROWS)],
                            o.at[pl.ds(tail, REM_ROWS)])

    @pl.when(cid == 0)
    def _():
        _writeback(o0)

    @pl.when(cid == 1)
    def _():
        _writeback(o1)


def kernel(x, edge_index, edge_values):
    x0 = x[:, :DH]
    x1 = x[:, DH:]
    row_r = edge_index[0].reshape(NS, NSUP, NB, CHUNK)
    col_r = edge_index[1].reshape(NS, NSUP, NB, CHUNK)
    w_r = edge_values

    f = pl.kernel(
        _body,
        out_type=(jax.ShapeDtypeStruct((N_NODES, DH), jnp.float32),
                  jax.ShapeDtypeStruct((N_NODES, DH), jnp.float32)),
        mesh=plsc.VectorSubcoreMesh(core_axis_name="c", subcore_axis_name="s",
                                    num_cores=NC, num_subcores=NS),
        compiler_params=pltpu.CompilerParams(needs_layout_passes=False,
                                             use_tc_tiling_on_sc=False),
        scratch_types=[
            pltpu.VMEM_SHARED((N_NODES, DH), jnp.float32),
            pltpu.VMEM_SHARED((N_NODES, DH), jnp.float32),
            pltpu.VMEM((NB, CHUNK), jnp.int32),
            pltpu.VMEM((NB, CHUNK), jnp.int32),
            pltpu.VMEM((NB * CHUNK,), jnp.float32),
            pltpu.VMEM((CHUNK, DH), jnp.float32),
            pltpu.VMEM((CHUNK, DH), jnp.float32),
            pltpu.VMEM((CHUNK, DH), jnp.float32),
            pltpu.VMEM((CHUNK, DH), jnp.float32),
            pltpu.SemaphoreType.DMA,
            pltpu.SemaphoreType.DMA,
            pltpu.SemaphoreType.DMA,
            pltpu.SemaphoreType.DMA,
        ],
    )
    o0, o1 = f(x0, x1, col_r, row_r, w_r)
    return jnp.concatenate([o0, o1], axis=1)


# speedup vs baseline: 9.8121x; 1.1411x over previous
import jax
import jax.numpy as jnp
from jax import lax
from jax.experimental import pallas as pl
from jax.experimental.pallas import tpu as pltpu
from jax.experimental.pallas import tpu_sc as plsc

N_NODES = 10000
N_EDGES = 320000
D = 128
DH = 64
NC = 2
NS = 16
L = 16
CHUNK = 80
NB = 50
EPT = N_EDGES // NS
NCH = EPT // CHUNK
NSUP = NCH // NB
ROWS_PT = 624
REM_ROWS = N_NODES - NS * ROWS_PT


def _zero_buf(buf):
    zeros = jnp.zeros((L,), jnp.float32)

    def zb(e, c):
        for d in range(DH // L):
            buf[e, pl.ds(d * L, L)] = zeros
        return c

    lax.fori_loop(0, CHUNK, zb, 0)


def _zero_acc(acc, zbuf, sid):
    r0 = pl.multiple_of(sid * ROWS_PT, 8)
    full = ROWS_PT // CHUNK
    rem = ROWS_PT - full * CHUNK
    for j in range(full):
        pltpu.sync_copy(zbuf, acc.at[pl.ds(r0 + j * CHUNK, CHUNK)])
    if rem:
        pltpu.sync_copy(zbuf.at[pl.ds(0, rem)],
                        acc.at[pl.ds(r0 + full * CHUNK, rem)])

    @pl.when(sid == NS - 1)
    def _():
        pltpu.sync_copy(zbuf.at[pl.ds(0, REM_ROWS)],
                        acc.at[pl.ds(NS * ROWS_PT, REM_ROWS)])


def _hop(src, dst, sid, col_r, row_r, w_hbm,
         colv, rowv, wv, gbufs, sbufs, gsems, ssems):
    def super_body(j, c0):
        pltpu.sync_copy(col_r.at[sid, j], colv)
        pltpu.sync_copy(row_r.at[sid, j], rowv)
        wbase = pl.multiple_of(sid * EPT + j * (NB * CHUNK), 8)
        pltpu.sync_copy(w_hbm.at[pl.ds(wbase, NB * CHUNK)], wv)

        for b in range(2):
            pltpu.async_copy(src.at[colv.at[b]], gbufs[b], gsems[b])

        def pair_body(k, c):
            for b in range(2):
                i = k * 2 + b
                pltpu.make_async_copy(src.at[colv.at[i]],
                                      gbufs[b], gsems[b]).wait()

                @pl.when(k >= 1)
                def _():
                    pltpu.make_async_copy(sbufs[b], dst.at[rowv.at[i - 2]],
                                          ssems[b]).wait()

                @plsc.parallel_loop(0, CHUNK, unroll=8)
                def scale(e):
                    wb = plsc.load_gather(
                        wv, [jnp.full((L,), i * CHUNK + e, jnp.int32)])
                    for d in range(DH // L):
                        sl = pl.ds(d * L, L)
                        sbufs[b][e, sl] = gbufs[b][e, sl] * wb

                pltpu.async_copy(sbufs[b], dst.at[rowv.at[i]], ssems[b],
                                 add=True)

                @pl.when(k < NB // 2 - 1)
                def _():
                    pltpu.async_copy(src.at[colv.at[i + 2]],
                                     gbufs[b], gsems[b])

            return c

        lax.fori_loop(0, NB // 2, pair_body, 0)

        for b in range(2):
            pltpu.make_async_copy(sbufs[b], dst.at[rowv.at[NB - 2 + b]],
                                  ssems[b]).wait()
        return c0

    lax.fori_loop(0, NSUP, super_body, 0)


def _body(x0, x1, col_r, row_r, w_hbm, o0, o1,
          acc1, acc2, colv, rowv, wv, gbuf0, gbuf1, sbuf0, sbuf1,
          gsem0, gsem1, ssem0, ssem1):
    cid = lax.axis_index("c")
    sid = lax.axis_index("s")
    gbufs = (gbuf0, gbuf1)
    sbufs = (sbuf0, sbuf1)
    gsems = (gsem0, gsem1)
    ssems = (ssem0, ssem1)

    _zero_buf(sbuf0)
    _zero_acc(acc1, sbuf0, sid)
    _zero_acc(acc2, sbuf0, sid)
    plsc.subcore_barrier()

    @pl.when(cid == 0)
    def _():
        _hop(x0, acc1, sid, col_r, row_r, w_hbm,
             colv, rowv, wv, gbufs, sbufs, gsems, ssems)

    @pl.when(cid == 1)
    def _():
        _hop(x1, acc1, sid, col_r, row_r, w_hbm,
             colv, rowv, wv, gbufs, sbufs, gsems, ssems)

    plsc.subcore_barrier()
    _hop(acc1, acc2, sid, col_r, row_r, w_hbm,
         colv, rowv, wv, gbufs, sbufs, gsems, ssems)
    plsc.subcore_barrier()

    r0 = pl.multiple_of(sid * ROWS_PT, 8)
    tail = NS * ROWS_PT

    def _writeback(o):
        pltpu.sync_copy(acc2.at[pl.ds(r0, ROWS_PT)], o.at[pl.ds(r0, ROWS_PT)])

        @pl.when(sid == NS - 1)
        def _():
            pltpu.sync_copy(acc2.at[pl.ds(tail, REM_ROWS)],
                            o.at[pl.ds(tail, REM_ROWS)])

    @pl.when(cid == 0)
    def _():
        _writeback(o0)

    @pl.when(cid == 1)
    def _():
        _writeback(o1)


def kernel(x, edge_index, edge_values):
    x0 = x[:, :DH]
    x1 = x[:, DH:]
    row_r = edge_index[0].reshape(NS, NSUP, NB, CHUNK)
    col_r = edge_index[1].reshape(NS, NSUP, NB, CHUNK)
    w_r = edge_values

    f = pl.kernel(
        _body,
        out_type=(jax.ShapeDtypeStruct((N_NODES, DH), jnp.float32),
                  jax.ShapeDtypeStruct((N_NODES, DH), jnp.float32)),
        mesh=plsc.VectorSubcoreMesh(core_axis_name="c", subcore_axis_name="s",
                                    num_cores=NC, num_subcores=NS),
        compiler_params=pltpu.CompilerParams(needs_layout_passes=False,
                                             use_tc_tiling_on_sc=False),
        scratch_types=[
            pltpu.VMEM_SHARED((N_NODES, DH), jnp.float32),
            pltpu.VMEM_SHARED((N_NODES, DH), jnp.float32),
            pltpu.VMEM((NB, CHUNK), jnp.int32),
            pltpu.VMEM((NB, CHUNK), jnp.int32),
            pltpu.VMEM((NB * CHUNK,), jnp.float32),
            pltpu.VMEM((CHUNK, DH), jnp.float32),
            pltpu.VMEM((CHUNK, DH), jnp.float32),
            pltpu.VMEM((CHUNK, DH), jnp.float32),
            pltpu.VMEM((CHUNK, DH), jnp.float32),
            pltpu.SemaphoreType.DMA,
            pltpu.SemaphoreType.DMA,
            pltpu.SemaphoreType.DMA,
            pltpu.SemaphoreType.DMA,
        ],
    )
    o0, o1 = f(x0, x1, col_r, row_r, w_r)
    return jnp.concatenate([o0, o1], axis=1)


# speedup vs baseline: 9.8455x; 1.0034x over previous
import jax
import jax.numpy as jnp
from jax import lax
from jax.experimental import pallas as pl
from jax.experimental.pallas import tpu as pltpu
from jax.experimental.pallas import tpu_sc as plsc

N_NODES = 10000
N_EDGES = 320000
D = 128
DH = 64
NC = 2
NS = 16
L = 16
CHUNK = 80
NB = 50
EPT = N_EDGES // NS
NCH = EPT // CHUNK
NSUP = NCH // NB
ROWS_PT = 624
REM_ROWS = N_NODES - NS * ROWS_PT


def _zero_buf(buf):
    zeros = jnp.zeros((L,), jnp.float32)

    def zb(e, c):
        for d in range(DH // L):
            buf[e, pl.ds(d * L, L)] = zeros
        return c

    lax.fori_loop(0, CHUNK, zb, 0)


def _zero_acc(acc, zbuf, sid):
    r0 = pl.multiple_of(sid * ROWS_PT, 8)
    full = ROWS_PT // CHUNK
    rem = ROWS_PT - full * CHUNK
    for j in range(full):
        pltpu.sync_copy(zbuf, acc.at[pl.ds(r0 + j * CHUNK, CHUNK)])
    if rem:
        pltpu.sync_copy(zbuf.at[pl.ds(0, rem)],
                        acc.at[pl.ds(r0 + full * CHUNK, rem)])

    @pl.when(sid == NS - 1)
    def _():
        pltpu.sync_copy(zbuf.at[pl.ds(0, REM_ROWS)],
                        acc.at[pl.ds(NS * ROWS_PT, REM_ROWS)])


def _hop(src, dst, sid, col_r, row_r, w_hbm,
         colv, rowv, wv, gbufs, sbufs, gsems, ssems):
    def super_body(j, c0):
        pltpu.sync_copy(col_r.at[sid, j], colv)
        pltpu.sync_copy(row_r.at[sid, j], rowv)
        wbase = pl.multiple_of(sid * EPT + j * (NB * CHUNK), 8)
        pltpu.sync_copy(w_hbm.at[pl.ds(wbase, NB * CHUNK)], wv)

        for b in range(2):
            pltpu.async_copy(src.at[colv.at[b]], gbufs[b], gsems[b])

        def pair_body(k, c):
            for b in range(2):
                i = k * 2 + b
                pltpu.make_async_copy(src.at[colv.at[i]],
                                      gbufs[b], gsems[b]).wait()

                @pl.when(k >= 1)
                def _():
                    pltpu.make_async_copy(sbufs[b], dst.at[rowv.at[i - 2]],
                                          ssems[b]).wait()

                @plsc.parallel_loop(0, CHUNK, unroll=8)
                def scale(e):
                    wb = plsc.load_gather(
                        wv, [jnp.full((L,), i * CHUNK + e, jnp.int32)])
                    for d in range(DH // L):
                        sl = pl.ds(d * L, L)
                        sbufs[b][e, sl] = gbufs[b][e, sl] * wb

                pltpu.async_copy(sbufs[b], dst.at[rowv.at[i]], ssems[b],
                                 add=True)

                @pl.when(k < NB // 2 - 1)
                def _():
                    pltpu.async_copy(src.at[colv.at[i + 2]],
                                     gbufs[b], gsems[b])

            return c

        lax.fori_loop(0, NB // 2, pair_body, 0)

        for b in range(2):
            pltpu.make_async_copy(sbufs[b], dst.at[rowv.at[NB - 2 + b]],
                                  ssems[b]).wait()
        return c0

    lax.fori_loop(0, NSUP, super_body, 0)


def _body(x0, x1, col_r, row_r, w_hbm, o0, o1, h0, h1,
          acc1, acc2, colv, rowv, wv, gbuf0, gbuf1, sbuf0, sbuf1,
          gsem0, gsem1, ssem0, ssem1):
    cid = lax.axis_index("c")
    sid = lax.axis_index("s")
    gbufs = (gbuf0, gbuf1)
    sbufs = (sbuf0, sbuf1)
    gsems = (gsem0, gsem1)
    ssems = (ssem0, ssem1)

    _zero_buf(sbuf0)
    _zero_acc(acc1, sbuf0, sid)
    _zero_acc(acc2, sbuf0, sid)
    plsc.subcore_barrier()

    @pl.when(cid == 0)
    def _():
        _hop(x0, acc1, sid, col_r, row_r, w_hbm,
             colv, rowv, wv, gbufs, sbufs, gsems, ssems)

    @pl.when(cid == 1)
    def _():
        _hop(x1, acc1, sid, col_r, row_r, w_hbm,
             colv, rowv, wv, gbufs, sbufs, gsems, ssems)

    plsc.subcore_barrier()

    r0 = pl.multiple_of(sid * ROWS_PT, 8)
    tail = NS * ROWS_PT

    def _writeback_from(acc, o):
        pltpu.sync_copy(acc.at[pl.ds(r0, ROWS_PT)], o.at[pl.ds(r0, ROWS_PT)])

        @pl.when(sid == NS - 1)
        def _():
            pltpu.sync_copy(acc.at[pl.ds(tail, REM_ROWS)],
                            o.at[pl.ds(tail, REM_ROWS)])

    @pl.when(cid == 0)
    def _():
        _writeback_from(acc1, h0)

    @pl.when(cid == 1)
    def _():
        _writeback_from(acc1, h1)

    plsc.subcore_barrier()

    @pl.when(cid == 0)
    def _():
        _hop(h0, acc2, sid, col_r, row_r, w_hbm,
             colv, rowv, wv, gbufs, sbufs, gsems, ssems)

    @pl.when(cid == 1)
    def _():
        _hop(h1, acc2, sid, col_r, row_r, w_hbm,
             colv, rowv, wv, gbufs, sbufs, gsems, ssems)

    plsc.subcore_barrier()

    def _writeback(o):
        _writeback_from(acc2, o)

    @pl.when(cid == 0)
    def _():
        _writeback(o0)

    @pl.when(cid == 1)
    def _():
        _writeback(o1)


def kernel(x, edge_index, edge_values):
    x0 = x[:, :DH]
    x1 = x[:, DH:]
    row_r = edge_index[0].reshape(NS, NSUP, NB, CHUNK)
    col_r = edge_index[1].reshape(NS, NSUP, NB, CHUNK)
    w_r = edge_values

    f = pl.kernel(
        _body,
        out_type=(jax.ShapeDtypeStruct((N_NODES, DH), jnp.float32),
                  jax.ShapeDtypeStruct((N_NODES, DH), jnp.float32),
                  jax.ShapeDtypeStruct((N_NODES, DH), jnp.float32),
                  jax.ShapeDtypeStruct((N_NODES, DH), jnp.float32)),
        mesh=plsc.VectorSubcoreMesh(core_axis_name="c", subcore_axis_name="s",
                                    num_cores=NC, num_subcores=NS),
        compiler_params=pltpu.CompilerParams(needs_layout_passes=False,
                                             use_tc_tiling_on_sc=False),
        scratch_types=[
            pltpu.VMEM_SHARED((N_NODES, DH), jnp.float32),
            pltpu.VMEM_SHARED((N_NODES, DH), jnp.float32),
            pltpu.VMEM((NB, CHUNK), jnp.int32),
            pltpu.VMEM((NB, CHUNK), jnp.int32),
            pltpu.VMEM((NB * CHUNK,), jnp.float32),
            pltpu.VMEM((CHUNK, DH), jnp.float32),
            pltpu.VMEM((CHUNK, DH), jnp.float32),
            pltpu.VMEM((CHUNK, DH), jnp.float32),
            pltpu.VMEM((CHUNK, DH), jnp.float32),
            pltpu.SemaphoreType.DMA,
            pltpu.SemaphoreType.DMA,
            pltpu.SemaphoreType.DMA,
            pltpu.SemaphoreType.DMA,
        ],
    )
    o0, o1, _, _ = f(x0, x1, col_r, row_r, w_r)
    return jnp.concatenate([o0, o1], axis=1)


# speedup vs baseline: 10.1489x; 1.0308x over previous
import jax
import jax.numpy as jnp
from jax import lax
from jax.experimental import pallas as pl
from jax.experimental.pallas import tpu as pltpu
from jax.experimental.pallas import tpu_sc as plsc

N_NODES = 10000
N_EDGES = 320000
D = 128
DH = 64
NC = 2
NS = 16
L = 16
CHUNK = 250
G = 1
NB = 16
EPT = N_EDGES // NS
NCH = EPT // CHUNK
NSUP = NCH // NB
NGS = NB // G
ROWS_PT = 624
REM_ROWS = N_NODES - NS * ROWS_PT


def _zero_buf(buf):
    zeros = jnp.zeros((L,), jnp.float32)

    def zb(e, c):
        for d in range(DH // L):
            buf[e, pl.ds(d * L, L)] = zeros
        return c

    lax.fori_loop(0, CHUNK, zb, 0)


def _zero_acc(acc, zbuf, sid):
    r0 = pl.multiple_of(sid * ROWS_PT, 8)
    full = ROWS_PT // CHUNK
    rem = ROWS_PT - full * CHUNK
    for j in range(full):
        pltpu.sync_copy(zbuf, acc.at[pl.ds(r0 + j * CHUNK, CHUNK)])
    if rem:
        pltpu.sync_copy(zbuf.at[pl.ds(0, rem)],
                        acc.at[pl.ds(r0 + full * CHUNK, rem)])

    @pl.when(sid == NS - 1)
    def _():
        pltpu.sync_copy(zbuf.at[pl.ds(0, REM_ROWS)],
                        acc.at[pl.ds(NS * ROWS_PT, REM_ROWS)])


def _hop(src, dst, sid, col_r, row_r, w_hbm,
         colv, rowv, wv, gbufs, sbufs, gsems, ssems):
    def super_body(j, c0):
        pltpu.sync_copy(col_r.at[sid, j], colv)
        pltpu.sync_copy(row_r.at[sid, j], rowv)
        wbase = pl.multiple_of(sid * EPT + j * (NB * CHUNK), 8)
        pltpu.sync_copy(w_hbm.at[pl.ds(wbase, NB * CHUNK)], wv)

        for b in range(2):
            pltpu.async_copy(src.at[colv.at[b]],
                             gbufs[b], gsems[b])

        def pair_body(k, c):
            for b in range(2):
                i = k * 2 + b
                pltpu.make_async_copy(src.at[colv.at[i]],
                                      gbufs[b], gsems[b]).wait()

                @pl.when(k >= 1)
                def _():
                    pltpu.make_async_copy(
                        sbufs[b], dst.at[rowv.at[i - 2]],
                        ssems[b]).wait()

                @plsc.parallel_loop(0, CHUNK, unroll=10)
                def scale(e):
                    wb = plsc.load_gather(
                        wv, [jnp.full((L,), i * CHUNK + e, jnp.int32)])
                    for d in range(DH // L):
                        sl = pl.ds(d * L, L)
                        sbufs[b][e, sl] = gbufs[b][e, sl] * wb

                pltpu.async_copy(sbufs[b], dst.at[rowv.at[i]],
                                 ssems[b], add=True)

                @pl.when(k < NGS // 2 - 1)
                def _():
                    pltpu.async_copy(src.at[colv.at[i + 2]],
                                     gbufs[b], gsems[b])

            return c

        lax.fori_loop(0, NGS // 2, pair_body, 0)

        for b in range(2):
            pltpu.make_async_copy(sbufs[b],
                                  dst.at[rowv.at[NGS - 2 + b]],
                                  ssems[b]).wait()
        return c0

    lax.fori_loop(0, NSUP, super_body, 0)


def _body(x0, x1, col_r, row_r, w_hbm, o0, o1, h0, h1,
          acc, colv, rowv, wv, gbuf0, gbuf1, sbuf0, sbuf1,
          gsem0, gsem1, ssem0, ssem1):
    cid = lax.axis_index("c")
    sid = lax.axis_index("s")
    gbufs = (gbuf0, gbuf1)
    sbufs = (sbuf0, sbuf1)
    gsems = (gsem0, gsem1)
    ssems = (ssem0, ssem1)

    r0 = pl.multiple_of(sid * ROWS_PT, 8)
    tail = NS * ROWS_PT

    def _writeback_from(a, o):
        pltpu.sync_copy(a.at[pl.ds(r0, ROWS_PT)], o.at[pl.ds(r0, ROWS_PT)])

        @pl.when(sid == NS - 1)
        def _():
            pltpu.sync_copy(a.at[pl.ds(tail, REM_ROWS)],
                            o.at[pl.ds(tail, REM_ROWS)])

    _zero_buf(sbuf0)
    _zero_acc(acc, sbuf0, sid)
    plsc.subcore_barrier()

    @pl.when(cid == 0)
    def _():
        _hop(x0, acc, sid, col_r, row_r, w_hbm,
             colv, rowv, wv, gbufs, sbufs, gsems, ssems)

    @pl.when(cid == 1)
    def _():
        _hop(x1, acc, sid, col_r, row_r, w_hbm,
             colv, rowv, wv, gbufs, sbufs, gsems, ssems)

    plsc.subcore_barrier()

    @pl.when(cid == 0)
    def _():
        _writeback_from(acc, h0)

    @pl.when(cid == 1)
    def _():
        _writeback_from(acc, h1)

    plsc.subcore_barrier()
    _zero_buf(sbuf0)
    _zero_acc(acc, sbuf0, sid)
    plsc.subcore_barrier()

    @pl.when(cid == 0)
    def _():
        _hop(h0, acc, sid, col_r, row_r, w_hbm,
             colv, rowv, wv, gbufs, sbufs, gsems, ssems)

    @pl.when(cid == 1)
    def _():
        _hop(h1, acc, sid, col_r, row_r, w_hbm,
             colv, rowv, wv, gbufs, sbufs, gsems, ssems)

    plsc.subcore_barrier()

    @pl.when(cid == 0)
    def _():
        _writeback_from(acc, o0)

    @pl.when(cid == 1)
    def _():
        _writeback_from(acc, o1)


def kernel(x, edge_index, edge_values):
    x0 = x[:, :DH]
    x1 = x[:, DH:]
    row_r = edge_index[0].reshape(NS, NSUP, NB, CHUNK)
    col_r = edge_index[1].reshape(NS, NSUP, NB, CHUNK)
    w_r = edge_values

    f = pl.kernel(
        _body,
        out_type=(jax.ShapeDtypeStruct((N_NODES, DH), jnp.float32),
                  jax.ShapeDtypeStruct((N_NODES, DH), jnp.float32),
                  jax.ShapeDtypeStruct((N_NODES, DH), jnp.float32),
                  jax.ShapeDtypeStruct((N_NODES, DH), jnp.float32)),
        mesh=plsc.VectorSubcoreMesh(core_axis_name="c", subcore_axis_name="s",
                                    num_cores=NC, num_subcores=NS),
        compiler_params=pltpu.CompilerParams(needs_layout_passes=False,
                                             use_tc_tiling_on_sc=False),
        scratch_types=[
            pltpu.VMEM_SHARED((N_NODES, DH), jnp.float32),
            pltpu.VMEM((NB, CHUNK), jnp.int32),
            pltpu.VMEM((NB, CHUNK), jnp.int32),
            pltpu.VMEM((NB * CHUNK,), jnp.float32),
            pltpu.VMEM((CHUNK, DH), jnp.float32),
            pltpu.VMEM((CHUNK, DH), jnp.float32),
            pltpu.VMEM((CHUNK, DH), jnp.float32),
            pltpu.VMEM((CHUNK, DH), jnp.float32),
            pltpu.SemaphoreType.DMA,
            pltpu.SemaphoreType.DMA,
            pltpu.SemaphoreType.DMA,
            pltpu.SemaphoreType.DMA,
        ],
    )
    o0, o1, _, _ = f(x0, x1, col_r, row_r, w_r)
    return jnp.concatenate([o0, o1], axis=1)


# speedup vs baseline: 11.1606x; 1.0997x over previous
import jax
import jax.numpy as jnp
from jax import lax
from jax.experimental import pallas as pl
from jax.experimental.pallas import tpu as pltpu
from jax.experimental.pallas import tpu_sc as plsc

N_NODES = 10000
N_EDGES = 320000
D = 128
DH = 64
NC = 2
NS = 16
L = 16
CHUNK = 200
NB = 20
NG = 4
EPT = N_EDGES // NS
NCH = EPT // CHUNK
NSUP = NCH // NB
ROWS_PT = 624
REM_ROWS = N_NODES - NS * ROWS_PT


def _zero_buf(buf):
    zeros = jnp.zeros((L,), jnp.float32)

    def zb(e, c):
        for d in range(DH // L):
            buf[e, pl.ds(d * L, L)] = zeros
        return c

    lax.fori_loop(0, CHUNK, zb, 0)


def _zero_acc(acc, zbuf, sid):
    r0 = pl.multiple_of(sid * ROWS_PT, 8)
    full = ROWS_PT // CHUNK
    rem = ROWS_PT - full * CHUNK
    for j in range(full):
        pltpu.sync_copy(zbuf, acc.at[pl.ds(r0 + j * CHUNK, CHUNK)])
    if rem:
        pltpu.sync_copy(zbuf.at[pl.ds(0, rem)],
                        acc.at[pl.ds(r0 + full * CHUNK, rem)])

    @pl.when(sid == NS - 1)
    def _():
        pltpu.sync_copy(zbuf.at[pl.ds(0, REM_ROWS)],
                        acc.at[pl.ds(NS * ROWS_PT, REM_ROWS)])


def _hop(src, dst, sid, col_r, row_r, w_hbm,
         colv, rowv, wv, gbufs, sbufs, gsems, ssems):
    def super_body(j, c0):
        pltpu.sync_copy(col_r.at[sid, j], colv)
        pltpu.sync_copy(row_r.at[sid, j], rowv)
        wbase = pl.multiple_of(sid * EPT + j * (NB * CHUNK), 8)
        pltpu.sync_copy(w_hbm.at[pl.ds(wbase, NB * CHUNK)], wv)

        for b in range(NG - 1):
            pltpu.async_copy(src.at[colv.at[b]], gbufs[b], gsems[b])

        def quad_body(k, c):
            for b in range(NG):
                i = k * NG + b
                sb = b % 2
                pltpu.make_async_copy(src.at[colv.at[i]],
                                      gbufs[b], gsems[b]).wait()

                @pl.when(i + NG - 1 < NB)
                def _():
                    pltpu.async_copy(src.at[colv.at[i + NG - 1]],
                                     gbufs[(b + NG - 1) % NG],
                                     gsems[(b + NG - 1) % NG])

                @pl.when(i >= 2)
                def _():
                    pltpu.make_async_copy(
                        sbufs[sb], dst.at[rowv.at[i - 2]], ssems[sb]).wait()

                @plsc.parallel_loop(0, CHUNK, unroll=8)
                def scale(e):
                    wb = plsc.load_gather(
                        wv, [jnp.full((L,), i * CHUNK + e, jnp.int32)])
                    for d in range(DH // L):
                        sl = pl.ds(d * L, L)
                        sbufs[sb][e, sl] = gbufs[b][e, sl] * wb

                pltpu.async_copy(sbufs[sb], dst.at[rowv.at[i]], ssems[sb],
                                 add=True)

            return c

        lax.fori_loop(0, NB // NG, quad_body, 0)

        for i in (NB - 2, NB - 1):
            pltpu.make_async_copy(sbufs[i % 2], dst.at[rowv.at[i]],
                                  ssems[i % 2]).wait()
        return c0

    lax.fori_loop(0, NSUP, super_body, 0)


def _body(x0, x1, col_r, row_r, w_hbm, o0, o1, h0, h1,
          acc, colv, rowv, wv, gbuf0, gbuf1, gbuf2, gbuf3, sbuf0, sbuf1,
          gsem0, gsem1, gsem2, gsem3, ssem0, ssem1):
    cid = lax.axis_index("c")
    sid = lax.axis_index("s")
    gbufs = (gbuf0, gbuf1, gbuf2, gbuf3)
    sbufs = (sbuf0, sbuf1)
    gsems = (gsem0, gsem1, gsem2, gsem3)
    ssems = (ssem0, ssem1)

    r0 = pl.multiple_of(sid * ROWS_PT, 8)
    tail = NS * ROWS_PT

    def _writeback_from(a, o):
        pltpu.sync_copy(a.at[pl.ds(r0, ROWS_PT)], o.at[pl.ds(r0, ROWS_PT)])

        @pl.when(sid == NS - 1)
        def _():
            pltpu.sync_copy(a.at[pl.ds(tail, REM_ROWS)],
                            o.at[pl.ds(tail, REM_ROWS)])

    _zero_buf(sbuf0)
    _zero_acc(acc, sbuf0, sid)
    plsc.subcore_barrier()

    @pl.when(cid == 0)
    def _():
        _hop(x0, acc, sid, col_r, row_r, w_hbm,
             colv, rowv, wv, gbufs, sbufs, gsems, ssems)

    @pl.when(cid == 1)
    def _():
        _hop(x1, acc, sid, col_r, row_r, w_hbm,
             colv, rowv, wv, gbufs, sbufs, gsems, ssems)

    plsc.subcore_barrier()

    @pl.when(cid == 0)
    def _():
        _writeback_from(acc, h0)

    @pl.when(cid == 1)
    def _():
        _writeback_from(acc, h1)

    plsc.subcore_barrier()
    _zero_buf(sbuf0)
    _zero_acc(acc, sbuf0, sid)
    plsc.subcore_barrier()

    @pl.when(cid == 0)
    def _():
        _hop(h0, acc, sid, col_r, row_r, w_hbm,
             colv, rowv, wv, gbufs, sbufs, gsems, ssems)

    @pl.when(cid == 1)
    def _():
        _hop(h1, acc, sid, col_r, row_r, w_hbm,
             colv, rowv, wv, gbufs, sbufs, gsems, ssems)

    plsc.subcore_barrier()

    @pl.when(cid == 0)
    def _():
        _writeback_from(acc, o0)

    @pl.when(cid == 1)
    def _():
        _writeback_from(acc, o1)


def kernel(x, edge_index, edge_values):
    x0 = x[:, :DH]
    x1 = x[:, DH:]
    row_r = edge_index[0].reshape(NS, NSUP, NB, CHUNK)
    col_r = edge_index[1].reshape(NS, NSUP, NB, CHUNK)
    w_r = edge_values

    f = pl.kernel(
        _body,
        out_type=(jax.ShapeDtypeStruct((N_NODES, DH), jnp.float32),
                  jax.ShapeDtypeStruct((N_NODES, DH), jnp.float32),
                  jax.ShapeDtypeStruct((N_NODES, DH), jnp.float32),
                  jax.ShapeDtypeStruct((N_NODES, DH), jnp.float32)),
        mesh=plsc.VectorSubcoreMesh(core_axis_name="c", subcore_axis_name="s",
                                    num_cores=NC, num_subcores=NS),
        compiler_params=pltpu.CompilerParams(needs_layout_passes=False,
                                             use_tc_tiling_on_sc=False),
        scratch_types=[
            pltpu.VMEM_SHARED((N_NODES, DH), jnp.float32),
            pltpu.VMEM((NB, CHUNK), jnp.int32),
            pltpu.VMEM((NB, CHUNK), jnp.int32),
            pltpu.VMEM((NB * CHUNK,), jnp.float32),
            pltpu.VMEM((CHUNK, DH), jnp.float32),
            pltpu.VMEM((CHUNK, DH), jnp.float32),
            pltpu.VMEM((CHUNK, DH), jnp.float32),
            pltpu.VMEM((CHUNK, DH), jnp.float32),
            pltpu.VMEM((CHUNK, DH), jnp.float32),
            pltpu.VMEM((CHUNK, DH), jnp.float32),
            pltpu.SemaphoreType.DMA,
            pltpu.SemaphoreType.DMA,
            pltpu.SemaphoreType.DMA,
            pltpu.SemaphoreType.DMA,
            pltpu.SemaphoreType.DMA,
            pltpu.SemaphoreType.DMA,
        ],
    )
    o0, o1, _, _ = f(x0, x1, col_r, row_r, w_r)
    return jnp.concatenate([o0, o1], axis=1)
